# Initial kernel scaffold; baseline (speedup 1.0000x reference)
import jax
import jax.numpy as jnp
from jax.experimental import pallas as pl


def kernel(node_ft, batch, num_graphs):
    raise NotImplementedError("write your pallas kernel here")


# speedup vs baseline: 1.3028x; 1.3028x over previous
import functools

import jax
import jax.numpy as jnp
from jax import lax
from jax.experimental import pallas as pl
from jax.experimental.pallas import tpu as pltpu
from jax.experimental.pallas import tpu_sc as plsc

_L = 16
_CH_ROWS = 80
_GRP = _CH_ROWS // _L


def _make_kernel(n_rows: int, n_cols: int, n_seg: int):
    info = plsc.get_sparse_core_info()
    nc, ns = info.num_cores, info.num_subcores
    half = n_cols // nc
    kc = half // _L
    assert half % 128 == 0
    assert n_rows % _CH_ROWS == 0
    n_chunks = n_rows // _CH_ROWS
    n_half = n_seg // 2
    seg_rows = n_half // ns

    mesh = plsc.VectorSubcoreMesh(core_axis_name="c", subcore_axis_name="s")

    @functools.partial(
        pl.kernel,
        mesh=mesh,
        out_type=jax.ShapeDtypeStruct((n_seg, n_cols), jnp.float32),
        scratch_types=[
            pltpu.VMEM((_CH_ROWS, half), jnp.float32),
            pltpu.VMEM((_CH_ROWS + _L,), jnp.int32),
            pltpu.VMEM((n_seg, half), jnp.float32),
            pltpu.VMEM_SHARED((ns, n_half, half), jnp.float32),
        ],
    )
    def _k(node_hbm, idx_hbm, out_hbm, buf, idxv, acc, partials):
        c = lax.axis_index("c")
        s = lax.axis_index("s")
        col0 = pl.multiple_of(c * half, 128)

        zero = jnp.zeros((_L,), jnp.float32)

        def _zrow(r, carry):
            for k in range(kc):
                acc[r, pl.ds(k * _L, _L)] = zero
            return carry

        lax.fori_loop(0, n_seg, _zrow, 0)

        n_mine = (n_chunks - s + ns - 1) // ns

        def _chunk_body(i, carry):
            chunk = s + i * ns
            r0 = chunk * _CH_ROWS
            pltpu.sync_copy(node_hbm.at[pl.ds(r0, _CH_ROWS), pl.ds(col0, half)], buf)
            pltpu.sync_copy(idx_hbm.at[pl.ds(r0, _CH_ROWS)], idxv.at[pl.ds(0, _CH_ROWS)])

            def _row_body(r, carry2):
                sj = idxv[pl.ds(r, _L)][0]
                for k in range(kc):
                    plsc.addupdate(
                        acc.at[sj, pl.ds(k * _L, _L)],
                        buf[r, pl.ds(k * _L, _L)],
                    )
                return carry2

            lax.fori_loop(0, _CH_ROWS, _row_body, 0)
            return carry

        lax.fori_loop(0, n_mine, _chunk_body, 0)

        obuf = buf.at[pl.ds(0, seg_rows)]
        cbuf = buf.at[pl.ds(seg_rows, seg_rows)]
        for h in range(2):
            pltpu.sync_copy(acc.at[pl.ds(h * n_half, n_half)], partials.at[s])
            plsc.subcore_barrier()

            def _zorow(r, carry):
                for k in range(kc):
                    obuf[r, pl.ds(k * _L, _L)] = zero
                return carry

            lax.fori_loop(0, seg_rows, _zorow, 0)

            def _tbody(t, carry):
                pltpu.sync_copy(partials.at[t, pl.ds(s * seg_rows, seg_rows)], cbuf)

                def _addrow(r, carry2):
                    for k in range(kc):
                        plsc.addupdate(
                            obuf.at[r, pl.ds(k * _L, _L)],
                            cbuf[r, pl.ds(k * _L, _L)],
                        )
                    return carry2

                lax.fori_loop(0, seg_rows, _addrow, 0)
                return carry

            lax.fori_loop(0, ns, _tbody, 0)

            pltpu.sync_copy(
                obuf,
                out_hbm.at[pl.ds(h * n_half + s * seg_rows, seg_rows), pl.ds(col0, half)],
            )
            plsc.subcore_barrier()

    return _k


def kernel(node_ft, batch, num_graphs):
    n_rows, n_cols = node_ft.shape
    seg = jnp.minimum(batch, num_graphs - 1).astype(jnp.int32)
    k = _make_kernel(n_rows, n_cols, 256)
    return k(node_ft, seg)


# speedup vs baseline: 2.5166x; 1.9316x over previous
import functools

import jax
import jax.numpy as jnp
from jax import lax
from jax.experimental import pallas as pl
from jax.experimental.pallas import tpu as pltpu
from jax.experimental.pallas import tpu_sc as plsc

_L = 16
_CH_ROWS = 80


def _make_kernel(n_rows: int, n_cols: int, n_seg: int):
    info = plsc.get_sparse_core_info()
    nc, ns = info.num_cores, info.num_subcores
    half = n_cols // nc
    kc = half // _L
    assert half % 128 == 0
    assert n_rows % _CH_ROWS == 0
    n_chunks = n_rows // _CH_ROWS
    n_half = n_seg // 2
    seg_rows = n_half // ns

    mesh = plsc.VectorSubcoreMesh(core_axis_name="c", subcore_axis_name="s")

    @functools.partial(
        pl.kernel,
        mesh=mesh,
        out_type=jax.ShapeDtypeStruct((n_seg, n_cols), jnp.float32),
        scratch_types=[
            pltpu.VMEM((_CH_ROWS, half), jnp.float32),
            pltpu.VMEM((_CH_ROWS + _L,), jnp.int32),
            pltpu.VMEM((n_seg, half), jnp.float32),
            pltpu.VMEM_SHARED((ns, n_half, half), jnp.float32),
        ],
    )
    def _k(node_hbm, idx_hbm, out_hbm, buf, idxv, acc, partials):
        c = lax.axis_index("c")
        s = lax.axis_index("s")
        col0 = pl.multiple_of(c * half, 128)

        zero = jnp.zeros((_L,), jnp.float32)
        col_iota = lax.iota(jnp.int32, _L)

        def _zrow(r, carry):
            for k in range(kc):
                acc[r, pl.ds(k * _L, _L)] = zero
            return carry

        lax.fori_loop(0, n_seg, _zrow, 0)

        n_mine = (n_chunks - s + ns - 1) // ns

        def _chunk_body(i, carry):
            chunk = s + i * ns
            r0 = chunk * _CH_ROWS
            pltpu.sync_copy(node_hbm.at[pl.ds(r0, _CH_ROWS), pl.ds(col0, half)], buf)
            pltpu.sync_copy(idx_hbm.at[pl.ds(r0, _CH_ROWS)], idxv.at[pl.ds(0, _CH_ROWS)])

            @plsc.parallel_loop(0, _CH_ROWS, unroll=2)
            def _row_body(r):
                sj = idxv[pl.ds(r, _L)][0]
                for k in range(kc):
                    plsc.addupdate(
                        acc.at[sj, pl.ds(k * _L, _L)],
                        buf[r, pl.ds(k * _L, _L)],
                    )

            return carry

        lax.fori_loop(0, n_mine, _chunk_body, 0)

        obuf = buf.at[pl.ds(0, seg_rows)]
        cbuf = buf.at[pl.ds(seg_rows, seg_rows)]
        for h in range(2):
            pltpu.sync_copy(acc.at[pl.ds(h * n_half, n_half)], partials.at[s])
            plsc.subcore_barrier()

            def _zorow(r, carry):
                for k in range(kc):
                    obuf[r, pl.ds(k * _L, _L)] = zero
                return carry

            lax.fori_loop(0, seg_rows, _zorow, 0)

            def _tbody(t, carry):
                pltpu.sync_copy(partials.at[t, pl.ds(s * seg_rows, seg_rows)], cbuf)

                def _addrow(r, carry2):
                    for k in range(kc):
                        plsc.addupdate(
                            obuf.at[r, pl.ds(k * _L, _L)],
                            cbuf[r, pl.ds(k * _L, _L)],
                        )
                    return carry2

                lax.fori_loop(0, seg_rows, _addrow, 0)
                return carry

            lax.fori_loop(0, ns, _tbody, 0)

            pltpu.sync_copy(
                obuf,
                out_hbm.at[pl.ds(h * n_half + s * seg_rows, seg_rows), pl.ds(col0, half)],
            )
            plsc.subcore_barrier()

    return _k


def kernel(node_ft, batch, num_graphs):
    n_rows, n_cols = node_ft.shape
    seg = jnp.minimum(batch, num_graphs - 1).astype(jnp.int32)
    k = _make_kernel(n_rows, n_cols, 256)
    return k(node_ft, seg)
